# Initial kernel scaffold; baseline (speedup 1.0000x reference)
import jax
import jax.numpy as jnp
from jax.experimental import pallas as pl


def kernel(x, w_ih, w_hh, b_ih, b_hh):
    raise NotImplementedError("write your pallas kernel here")


# speedup vs baseline: 29.1659x; 29.1659x over previous
import jax
import jax.numpy as jnp
from jax.experimental import pallas as pl
from jax.experimental.pallas import tpu as pltpu


def _rnn_kernel(s_ref, x_ref, o_ref):
    j = pl.program_id(1)
    a = s_ref[0]
    b = s_ref[1]
    c = s_ref[2]

    @pl.when(j == 0)
    def _init():
        o_ref[...] = jnp.zeros_like(o_ref)

    tt = x_ref.shape[0]
    unroll = 8

    def body(i, h):
        base = i * unroll
        for k in range(unroll):
            h = jnp.tanh(x_ref[base + k] * a + (h * b + c))
        return h

    h = jax.lax.fori_loop(0, tt // unroll, body, o_ref[...])
    o_ref[...] = h


def kernel(x, w_ih, w_hh, b_ih, b_hh):
    B, T, _ = x.shape
    S = 8
    L = B // S
    NB = 2
    BL = L // NB
    TT = 512
    NT = T // TT

    x3 = x.reshape(B, T).T.reshape(T, S, L)
    scal = jnp.stack([w_ih[0, 0], w_hh[0, 0], b_ih[0] + b_hh[0]])

    out = pl.pallas_call(
        _rnn_kernel,
        grid=(NB, NT),
        in_specs=[
            pl.BlockSpec(memory_space=pltpu.SMEM),
            pl.BlockSpec((TT, S, BL), lambda i, j: (j, 0, i)),
        ],
        out_specs=pl.BlockSpec((S, BL), lambda i, j: (0, i)),
        out_shape=jax.ShapeDtypeStruct((S, L), x.dtype),
        compiler_params=pltpu.CompilerParams(
            dimension_semantics=("parallel", "arbitrary"),
        ),
    )(scal, x3)

    return out.reshape(B, 1)


# speedup vs baseline: 32.4597x; 1.1129x over previous
import jax
import jax.numpy as jnp
from jax.experimental import pallas as pl
from jax.experimental.pallas import tpu as pltpu


def _rnn_kernel(s_ref, x_ref, o_ref):
    j = pl.program_id(1)
    a = s_ref[0]
    b = s_ref[1]
    c = s_ref[2]

    @pl.when(j == 0)
    def _init():
        o_ref[...] = jnp.zeros_like(o_ref)

    tt = x_ref.shape[0]
    unroll = 16

    def body(i, h):
        base = i * unroll
        for k in range(unroll):
            u = x_ref[base + k] * a + c
            h = jnp.tanh(h * b + u)
        return h

    h = jax.lax.fori_loop(0, tt // unroll, body, o_ref[...])
    o_ref[...] = h


def kernel(x, w_ih, w_hh, b_ih, b_hh):
    B, T, _ = x.shape
    S = 8
    L = B // S
    NB = 2
    BL = L // NB
    TT = 512
    NT = T // TT

    x3 = jnp.transpose(x.reshape(S, L, T), (2, 0, 1))
    scal = jnp.stack([w_ih[0, 0], w_hh[0, 0], b_ih[0] + b_hh[0]])

    out = pl.pallas_call(
        _rnn_kernel,
        grid=(NB, NT),
        in_specs=[
            pl.BlockSpec(memory_space=pltpu.SMEM),
            pl.BlockSpec((TT, S, BL), lambda i, j: (j, 0, i)),
        ],
        out_specs=pl.BlockSpec((S, BL), lambda i, j: (0, i)),
        out_shape=jax.ShapeDtypeStruct((S, L), x.dtype),
        compiler_params=pltpu.CompilerParams(
            dimension_semantics=("parallel", "arbitrary"),
        ),
    )(scal, x3)

    return out.reshape(B, 1)


# speedup vs baseline: 37.7969x; 1.1644x over previous
import jax
import jax.numpy as jnp
from jax.experimental import pallas as pl
from jax.experimental.pallas import tpu as pltpu


def _rnn_kernel(s_ref, x_ref, o_ref, xs_ref):
    j = pl.program_id(1)
    a = s_ref[0]
    b = s_ref[1]
    c = s_ref[2]

    @pl.when(j == 0)
    def _init():
        o_ref[...] = jnp.zeros_like(o_ref)

    xs_ref[...] = jnp.transpose(x_ref[...], (2, 0, 1))

    tt = xs_ref.shape[0]
    unroll = 16

    def body(i, h):
        base = i * unroll
        for k in range(unroll):
            u = xs_ref[base + k] * a + c
            h = jnp.tanh(h * b + u)
        return h

    h = jax.lax.fori_loop(0, tt // unroll, body, o_ref[...])
    o_ref[...] = h


def kernel(x, w_ih, w_hh, b_ih, b_hh):
    B, T, _ = x.shape
    S = 8
    L = B // S
    NB = 2
    BL = L // NB
    TT = 512
    NT = T // TT

    x4 = x.reshape(S, L, T)
    scal = jnp.stack([w_ih[0, 0], w_hh[0, 0], b_ih[0] + b_hh[0]])

    out = pl.pallas_call(
        _rnn_kernel,
        grid=(NB, NT),
        in_specs=[
            pl.BlockSpec(memory_space=pltpu.SMEM),
            pl.BlockSpec((S, BL, TT), lambda i, j: (0, i, j)),
        ],
        out_specs=pl.BlockSpec((S, BL), lambda i, j: (0, i)),
        out_shape=jax.ShapeDtypeStruct((S, L), x.dtype),
        scratch_shapes=[pltpu.VMEM((TT, S, BL), x.dtype)],
        compiler_params=pltpu.CompilerParams(
            dimension_semantics=("parallel", "arbitrary"),
        ),
    )(scal, x4)

    return out.reshape(B, 1)


# speedup vs baseline: 37.8410x; 1.0012x over previous
import jax
import jax.numpy as jnp
from jax.experimental import pallas as pl
from jax.experimental.pallas import tpu as pltpu


def _rnn_kernel(s_ref, x_ref, o_ref, xs_ref):
    j = pl.program_id(1)
    a = s_ref[0]
    b = s_ref[1]
    c = s_ref[2]

    @pl.when(j == 0)
    def _init():
        o_ref[...] = jnp.zeros_like(o_ref)

    bb, tt = x_ref.shape
    bl = xs_ref.shape[2]

    xs_ref[...] = jnp.transpose(x_ref[...].reshape(8, bl, tt), (2, 0, 1))

    unroll = 16

    def body(i, h):
        base = i * unroll
        for k in range(unroll):
            u = xs_ref[base + k] * a + c
            h = jnp.tanh(h * b + u)
        return h

    h = jax.lax.fori_loop(0, tt // unroll, body, o_ref[0])
    o_ref[0] = h


def kernel(x, w_ih, w_hh, b_ih, b_hh):
    B, T, _ = x.shape
    NB = 2
    BB = B // NB
    BL = BB // 8
    TT = 512
    NT = T // TT

    x2 = x.reshape(B, T)
    scal = jnp.stack([w_ih[0, 0], w_hh[0, 0], b_ih[0] + b_hh[0]])

    out = pl.pallas_call(
        _rnn_kernel,
        grid=(NB, NT),
        in_specs=[
            pl.BlockSpec(memory_space=pltpu.SMEM),
            pl.BlockSpec((BB, TT), lambda i, j: (i, j)),
        ],
        out_specs=pl.BlockSpec((1, 8, BL), lambda i, j: (i, 0, 0)),
        out_shape=jax.ShapeDtypeStruct((NB, 8, BL), x.dtype),
        scratch_shapes=[pltpu.VMEM((TT, 8, BL), x.dtype)],
        compiler_params=pltpu.CompilerParams(
            dimension_semantics=("parallel", "arbitrary"),
        ),
    )(scal, x2)

    return out.reshape(B, 1)


# speedup vs baseline: 53.6407x; 1.4175x over previous
import jax
import jax.numpy as jnp
from jax.experimental import pallas as pl
from jax.experimental.pallas import tpu as pltpu


def _rnn_kernel(s_ref, x_ref, o_ref, xs_ref):
    j = pl.program_id(1)
    a = s_ref[0]
    b = s_ref[1]
    c = s_ref[2]

    @pl.when(j == 0)
    def _init():
        o_ref[...] = jnp.zeros_like(o_ref)

    n_k = x_ref.shape[1]
    bl = xs_ref.shape[3]
    unroll = 16

    h = o_ref[0]
    for p in range(n_k):
        w = x_ref[:, p, :].reshape(8, bl, 128)
        xs_ref[p % 2] = jnp.transpose(w, (2, 0, 1))

        def body(i, h, _p=p):
            base = i * unroll
            for k in range(unroll):
                u = xs_ref[_p % 2, base + k] * a + c
                h = jnp.tanh(h * b + u)
            return h

        h = jax.lax.fori_loop(0, 128 // unroll, body, h)
    o_ref[0] = h


def kernel(x, w_ih, w_hh, b_ih, b_hh):
    B, T, _ = x.shape
    NB = 2
    BB = B // NB
    BL = BB // 8
    TT = 1024
    NT = T // TT

    xv = x.reshape(B, T // 128, 128)
    scal = jnp.stack([w_ih[0, 0], w_hh[0, 0], b_ih[0] + b_hh[0]])

    out = pl.pallas_call(
        _rnn_kernel,
        grid=(NB, NT),
        in_specs=[
            pl.BlockSpec(memory_space=pltpu.SMEM),
            pl.BlockSpec((BB, TT // 128, 128), lambda i, j: (i, j, 0)),
        ],
        out_specs=pl.BlockSpec((1, 8, BL), lambda i, j: (i, 0, 0)),
        out_shape=jax.ShapeDtypeStruct((NB, 8, BL), x.dtype),
        scratch_shapes=[pltpu.VMEM((2, 128, 8, BL), x.dtype)],
        compiler_params=pltpu.CompilerParams(
            dimension_semantics=("parallel", "arbitrary"),
            vmem_limit_bytes=56 * 1024 * 1024,
        ),
    )(scal, xv)

    return out.reshape(B, 1)


# speedup vs baseline: 61.1004x; 1.1391x over previous
import jax
import jax.numpy as jnp
from jax.experimental import pallas as pl
from jax.experimental.pallas import tpu as pltpu


def _rnn_kernel(s_ref, x_ref, o_ref, xs_ref):
    j = pl.program_id(1)
    a = s_ref[0]
    b = s_ref[1]
    c = s_ref[2]

    @pl.when(j == 0)
    def _init():
        o_ref[...] = jnp.zeros_like(o_ref)

    n_k = x_ref.shape[1]
    bl = xs_ref.shape[3]

    h = o_ref[0]
    for p in range(n_k):
        w = x_ref[:, p, :].reshape(8, bl, 128)
        xs_ref[p % 2] = jnp.transpose(w, (2, 0, 1))
        for k in range(128):
            u = xs_ref[p % 2, k] * a + c
            h = jnp.tanh(h * b + u)
    o_ref[0] = h


def kernel(x, w_ih, w_hh, b_ih, b_hh):
    B, T, _ = x.shape
    NB = 2
    BB = B // NB
    BL = BB // 8
    TT = 1024
    NT = T // TT

    xv = x.reshape(B, T // 128, 128)
    scal = jnp.stack([w_ih[0, 0], w_hh[0, 0], b_ih[0] + b_hh[0]])

    out = pl.pallas_call(
        _rnn_kernel,
        grid=(NB, NT),
        in_specs=[
            pl.BlockSpec(memory_space=pltpu.SMEM),
            pl.BlockSpec((BB, TT // 128, 128), lambda i, j: (i, j, 0)),
        ],
        out_specs=pl.BlockSpec((1, 8, BL), lambda i, j: (i, 0, 0)),
        out_shape=jax.ShapeDtypeStruct((NB, 8, BL), x.dtype),
        scratch_shapes=[pltpu.VMEM((2, 128, 8, BL), x.dtype)],
        compiler_params=pltpu.CompilerParams(
            dimension_semantics=("parallel", "arbitrary"),
            vmem_limit_bytes=56 * 1024 * 1024,
        ),
    )(scal, xv)

    return out.reshape(B, 1)


# speedup vs baseline: 96.9573x; 1.5869x over previous
import jax
import jax.numpy as jnp
from jax.experimental import pallas as pl
from jax.experimental.pallas import tpu as pltpu


def _rnn_kernel(s_ref, x_hbm, o_ref, buf_ref, xs_ref, sem_ref):
    j = pl.program_id(0)
    n_p = pl.num_programs(0) - 1
    a = s_ref[0]
    b = s_ref[1]
    c = s_ref[2]
    bl = xs_ref.shape[3]

    @pl.when(j == 0)
    def _prologue():
        o_ref[...] = jnp.zeros_like(o_ref)
        for d in range(3):
            pltpu.make_async_copy(
                x_hbm.at[:, d, :], buf_ref.at[d], sem_ref.at[d]
            ).start()

    @pl.when(j < n_p)
    def _relayout():
        slot = j % 3
        pltpu.make_async_copy(
            x_hbm.at[:, j, :], buf_ref.at[slot], sem_ref.at[slot]
        ).wait()
        v = buf_ref[slot].reshape(8, bl, 128)
        xs_ref[j % 2] = jnp.transpose(v, (2, 0, 1))
        nxt = j + 3

        @pl.when(nxt < n_p)
        def _prefetch():
            pltpu.make_async_copy(
                x_hbm.at[:, nxt, :], buf_ref.at[slot], sem_ref.at[slot]
            ).start()

    @pl.when(j > 0)
    def _recurrence():
        rslot = (j - 1) % 2
        h = o_ref[...]
        for k in range(128):
            u = xs_ref[rslot, k] * a + c
            h = jnp.tanh(h * b + u)
        o_ref[...] = h


def kernel(x, w_ih, w_hh, b_ih, b_hh):
    B, T, _ = x.shape
    BL = B // 8
    NP = T // 128

    xv = x.reshape(B, T // 128, 128)
    scal = jnp.stack([w_ih[0, 0], w_hh[0, 0], b_ih[0] + b_hh[0]])

    out = pl.pallas_call(
        _rnn_kernel,
        grid=(NP + 1,),
        in_specs=[
            pl.BlockSpec(memory_space=pltpu.SMEM),
            pl.BlockSpec(memory_space=pl.ANY),
        ],
        out_specs=pl.BlockSpec((8, BL), lambda j: (0, 0)),
        out_shape=jax.ShapeDtypeStruct((8, BL), x.dtype),
        scratch_shapes=[
            pltpu.VMEM((3, B, 128), x.dtype),
            pltpu.VMEM((2, 128, 8, BL), x.dtype),
            pltpu.SemaphoreType.DMA((3,)),
        ],
        compiler_params=pltpu.CompilerParams(
            dimension_semantics=("arbitrary",),
            vmem_limit_bytes=56 * 1024 * 1024,
        ),
    )(scal, xv)

    return out.reshape(B, 1)
